# Initial kernel scaffold; baseline (speedup 1.0000x reference)
import jax
import jax.numpy as jnp
from jax.experimental import pallas as pl


def kernel(x, keys_p, values, Wqd, bqd, Wqu, Wvp):
    raise NotImplementedError("write your pallas kernel here")


# speedup vs baseline: 23.9223x; 23.9223x over previous
import functools

import jax
import jax.numpy as jnp
import numpy as np
from jax import lax
from jax.experimental import pallas as pl
from jax.experimental.pallas import tpu as pltpu
from jax.experimental.pallas import tpu_sc as plsc

_D_MODEL = 1024
_NKEYS = 512
_TOTAL = _NKEYS * _NKEYS
_NB = 18
_BD = 16
_KD = _NB * _BD
_H = 4
_KNN = 32
_QR = 128
_VD = 256
_SEQ = 2048
_ROWS = _SEQ * _H


def _roll_up(x, s):
    return jnp.concatenate([x[s:], x[:s]], axis=0)


def _roll_dn(x, s):
    return jnp.concatenate([x[-s:], x[:-s]], axis=0)


def _cmpex(pen, msk, s, iota_s, asc):
    low = (iota_s & s) == 0
    pen_p = jnp.where(low, _roll_up(pen, s), _roll_dn(pen, s))
    msk_p = jnp.where(low, _roll_up(msk, s), _roll_dn(msk, s))
    take_min = asc == low
    keep_self = ((take_min & (pen <= pen_p))
                 | (jnp.logical_not(take_min) & (pen >= pen_p)))
    return jnp.where(keep_self, pen, pen_p), jnp.where(keep_self, msk, msk_p)


def _tc_query_beam(xT_ref, ktm_ref, wqd_ref, bqd_ref, wqu_ref, idx_ref,
                   wbc_ref, dscr):
    f32 = jnp.float32
    hT = lax.dot_general(wqd_ref[...], xT_ref[...], (((1,), (0,)), ((), ())),
                         preferred_element_type=f32) + bqd_ref[...]
    qT = lax.dot_general(wqu_ref[...], hT, (((1,), (0,)), ((), ())),
                         preferred_element_type=f32)
    sT = lax.dot_general(ktm_ref[...], qT, (((1,), (0,)), ((), ())),
                         preferred_element_type=f32)
    s0 = sT[:72]
    s1 = sT[72:]
    delta = jnp.abs(s0 - s1)
    bits = (s1 > s0).astype(f32)
    mx = jnp.maximum(s0, s1)

    pow2 = jnp.left_shift(
        jnp.int32(1), lax.broadcasted_iota(jnp.int32, (_NB, 1), 0)
    ).astype(f32)
    codes = jnp.concatenate(
        [jnp.sum(bits[h * _NB:(h + 1) * _NB] * pow2, axis=0, keepdims=True)
         for h in range(_H)], axis=1)
    best = jnp.concatenate(
        [jnp.sum(mx[h * _NB:(h + 1) * _NB], axis=0, keepdims=True)
         for h in range(_H)], axis=1)
    dstep = [jnp.concatenate(
        [delta[h * _NB + t:h * _NB + t + 1] for h in range(_H)], axis=1)
        for t in range(_NB)]
    dscr[pl.ds(0, _NB), :] = jnp.concatenate(dstep, axis=0)

    iota_s = lax.broadcasted_iota(jnp.int32, (_KNN, 1), 0)

    pen = jnp.zeros((_KNN, _ROWS), f32)
    for t in range(5):
        bit = ((iota_s >> t) & 1) == 1
        pen = pen + jnp.where(bit, dstep[t], jnp.zeros((1, _ROWS), f32))
    msk = lax.broadcasted_iota(jnp.int32, (_KNN, _ROWS), 0).astype(f32)
    for k in (2, 4, 8, 16, 32):
        asc = (iota_s & k) == 0
        s = k // 2
        while s >= 1:
            pen, msk = _cmpex(pen, msk, s, iota_s, asc)
            s //= 2

    rev_p = (lax.broadcasted_iota(jnp.int32, (_KNN, _KNN), 0)
             + lax.broadcasted_iota(jnp.int32, (_KNN, _KNN), 1)
             == _KNN - 1).astype(f32)

    def _rev32(z):
        return lax.dot_general(rev_p, z, (((1,), (0,)), ((), ())),
                               preferred_element_type=f32,
                         precision=lax.Precision.HIGHEST)

    asc_all = jnp.full((_KNN, 1), True)

    def _merge_step(t, carry):
        pen, msk, bitf = carry
        rp = _rev32(pen) + dscr[pl.ds(t, 1), :]
        rm = _rev32(msk) + bitf
        keep = pen <= rp
        pen = jnp.where(keep, pen, rp)
        msk = jnp.where(keep, msk, rm)
        for s in (16, 8, 4, 2, 1):
            pen, msk = _cmpex(pen, msk, s, iota_s, asc_all)
        return pen, msk, bitf * f32(2.0)

    pen, msk, _ = lax.fori_loop(
        5, _NB, _merge_step, (pen, msk, f32(float(1 << 5))))

    sc = best - pen
    sc = sc - jnp.max(sc, axis=0, keepdims=True)
    e = jnp.exp(sc)
    w = e / jnp.sum(e, axis=0, keepdims=True)
    idx = codes.astype(jnp.int32) ^ msk.astype(jnp.int32)

    idx_cat = jnp.concatenate(
        [idx[:, h * _SEQ:(h + 1) * _SEQ] for h in range(_H)], axis=0)
    w_cat = jnp.concatenate(
        [w[:, h * _SEQ:(h + 1) * _SEQ] for h in range(_H)], axis=0)
    idx_ref[...] = jnp.swapaxes(idx_cat, 0, 1)
    wT = jnp.swapaxes(w_cat, 0, 1)
    rep = (lax.broadcasted_iota(jnp.int32, (_KPT, _LANES * _KPT), 1) // _LANES
           == lax.broadcasted_iota(jnp.int32, (_KPT, _LANES * _KPT), 0)
           ).astype(f32)
    wbc_ref[...] = lax.dot_general(wT, rep, (((1,), (0,)), ((), ())),
                                   preferred_element_type=f32,
                         precision=lax.Precision.HIGHEST)


def _run_tc_a(xT, ktm, wqd, bqd2, wqu):
    return pl.pallas_call(
        _tc_query_beam,
        out_shape=[
            jax.ShapeDtypeStruct((_SEQ, _KPT), jnp.int32),
            jax.ShapeDtypeStruct((_SEQ, _LANES * _KPT), jnp.float32),
        ],
        scratch_shapes=[pltpu.VMEM((24, _ROWS), jnp.float32)],
    )(xT, ktm, wqd, bqd2, wqu)


_LANES = 16
_NW = 32
_TPW = _SEQ // _NW
_KPT = _H * _KNN


_WGRP = 8


def _sc_bag_body(idx_hbm, wbc_hbm, val_hbm, out_hbm,
                 idxblk, wbuf, rows, outacc, sem_r0, sem_r1, sem_w0, sem_w1):
    f32 = jnp.float32
    wid = lax.axis_index("s") * 2 + lax.axis_index("c")
    base = wid * _TPW
    pltpu.sync_copy(idx_hbm.at[pl.ds(base, _TPW), :], idxblk)

    sem_r = (sem_r0, sem_r1)
    sem_w = (sem_w0, sem_w1)

    def start_rows(b, t):
        pltpu.make_async_copy(
            val_hbm.at[idxblk.at[t]], rows.at[b], sem_r[b]).start()

    def wait_rows(b):
        pltpu.make_async_copy(
            val_hbm.at[idxblk.at[0]], rows.at[b], sem_r[b]).wait()

    def start_wgrp(sub, g):
        pltpu.make_async_copy(
            wbc_hbm.at[pl.ds(base + g * _WGRP, _WGRP), :],
            wbuf.at[sub], sem_w[sub]).start()

    def wait_wgrp(sub):
        pltpu.make_async_copy(
            wbc_hbm.at[pl.ds(base, _WGRP), :],
            wbuf.at[sub], sem_w[sub]).wait()

    def accumulate(sub, i, b, t):
        def kbody(k, accs):
            wb = wbuf[sub, i, pl.ds(k * _LANES, _LANES)]
            return tuple(
                accs[j] + wb * rows[b, k, pl.ds(j * _LANES, _LANES)]
                for j in range(_VD // _LANES))
        accs = lax.fori_loop(
            0, _KPT, kbody,
            tuple(jnp.zeros((_LANES,), f32) for _ in range(_VD // _LANES)))
        for j in range(_VD // _LANES):
            outacc[t, pl.ds(j * _LANES, _LANES)] = accs[j]

    start_wgrp(0, 0)
    start_wgrp(1, 1)
    start_rows(0, 0)
    start_rows(1, 1)

    def gpbody(gp, carry):
        for sub in range(2):
            g = gp * 2 + sub
            wait_wgrp(sub)
            for i in range(_WGRP):
                t = g * _WGRP + i
                b = i % 2
                wait_rows(b)
                accumulate(sub, i, b, t)

                @pl.when(t + 2 < _TPW)
                def _pf():
                    start_rows(b, t + 2)

            @pl.when(g + 2 < _TPW // _WGRP)
            def _pfw():
                start_wgrp(sub, g + 2)
        return carry

    lax.fori_loop(0, _TPW // (2 * _WGRP), gpbody, 0)
    pltpu.sync_copy(outacc, out_hbm.at[pl.ds(base, _TPW), :])


def _run_sc_bag(idx, wbc, values):
    mesh = plsc.VectorSubcoreMesh(core_axis_name="c", subcore_axis_name="s")
    kern = functools.partial(
        pl.kernel,
        mesh=mesh,
        out_type=jax.ShapeDtypeStruct((_SEQ, _VD), jnp.float32),
        scratch_types=[
            pltpu.VMEM((_TPW, _KPT), jnp.int32),
            pltpu.VMEM((2, _WGRP, _LANES * _KPT), jnp.float32),
            pltpu.VMEM((2, _KPT, _VD), jnp.float32),
            pltpu.VMEM((_TPW, _VD), jnp.float32),
            pltpu.SemaphoreType.DMA,
            pltpu.SemaphoreType.DMA,
            pltpu.SemaphoreType.DMA,
            pltpu.SemaphoreType.DMA,
        ],
    )(_sc_bag_body)
    return kern(idx, wbc, values)


def _tc_proj_body(y0_ref, wvp_ref, o_ref):
    o_ref[...] = lax.dot_general(
        y0_ref[...], wvp_ref[...], (((1,), (1,)), ((), ())),
        preferred_element_type=jnp.float32)


def _run_tc_c(y0, wvp):
    return pl.pallas_call(
        _tc_proj_body,
        out_shape=jax.ShapeDtypeStruct((_SEQ, _D_MODEL), jnp.float32),
    )(y0, wvp)


_COLJ = np.arange(_KD * _H) // _BD
_ROWJ = np.arange(2 * _H * _NB) % (_H * _NB)
_KMASK = (_ROWJ[:, None] == _COLJ[None, :]).astype(np.float32)


def kernel(x, keys_p, values, Wqd, bqd, Wqu, Wvp):
    bsz, seq_len, _ = x.shape
    xT = jnp.swapaxes(x.reshape(bsz * seq_len, _D_MODEL), 0, 1)
    keys_r = jnp.transpose(keys_p, (2, 0, 1, 3)).reshape(2 * _H * _NB, _BD)
    ktm = jnp.tile(keys_r, (1, _H * _NB)) * _KMASK
    bqd2 = bqd.reshape(_QR, 1)
    idx, wbc = _run_tc_a(xT, ktm, Wqd, bqd2, Wqu)
    y0 = _run_sc_bag(idx, wbc, values)
    y = _run_tc_c(y0, Wvp)
    return y.reshape(bsz, seq_len, _D_MODEL)


# speedup vs baseline: 24.0491x; 1.0053x over previous
import functools

import jax
import jax.numpy as jnp
import numpy as np
from jax import lax
from jax.experimental import pallas as pl
from jax.experimental.pallas import tpu as pltpu
from jax.experimental.pallas import tpu_sc as plsc

_D_MODEL = 1024
_NKEYS = 512
_TOTAL = _NKEYS * _NKEYS
_NB = 18
_BD = 16
_KD = _NB * _BD
_H = 4
_KNN = 32
_QR = 128
_VD = 256
_SEQ = 2048
_ROWS = _SEQ * _H


def _roll_up(x, s):
    return jnp.concatenate([x[s:], x[:s]], axis=0)


def _roll_dn(x, s):
    return jnp.concatenate([x[-s:], x[:-s]], axis=0)


def _cmpex(pen, msk, s, iota_s, asc):
    low = (iota_s & s) == 0
    pen_p = jnp.where(low, _roll_up(pen, s), _roll_dn(pen, s))
    msk_p = jnp.where(low, _roll_up(msk, s), _roll_dn(msk, s))
    take_min = asc == low
    keep_self = ((take_min & (pen <= pen_p))
                 | (jnp.logical_not(take_min) & (pen >= pen_p)))
    return jnp.where(keep_self, pen, pen_p), jnp.where(keep_self, msk, msk_p)


def _tc_query_beam(xT_ref, ktm_ref, wqd_ref, bqd_ref, wqu_ref, idx_ref,
                   wbc_ref, dscr):
    f32 = jnp.float32
    hT = lax.dot_general(wqd_ref[...], xT_ref[...], (((1,), (0,)), ((), ())),
                         preferred_element_type=f32) + bqd_ref[...]
    qT = lax.dot_general(wqu_ref[...], hT, (((1,), (0,)), ((), ())),
                         preferred_element_type=f32)
    sT = lax.dot_general(ktm_ref[...], qT, (((1,), (0,)), ((), ())),
                         preferred_element_type=f32)
    s0 = sT[:72]
    s1 = sT[72:]
    delta = jnp.abs(s0 - s1)
    bits = (s1 > s0).astype(f32)
    mx = jnp.maximum(s0, s1)

    pow2 = jnp.left_shift(
        jnp.int32(1), lax.broadcasted_iota(jnp.int32, (_NB, 1), 0)
    ).astype(f32)
    codes = jnp.concatenate(
        [jnp.sum(bits[h * _NB:(h + 1) * _NB] * pow2, axis=0, keepdims=True)
         for h in range(_H)], axis=1)
    best = jnp.concatenate(
        [jnp.sum(mx[h * _NB:(h + 1) * _NB], axis=0, keepdims=True)
         for h in range(_H)], axis=1)
    dstep = [jnp.concatenate(
        [delta[h * _NB + t:h * _NB + t + 1] for h in range(_H)], axis=1)
        for t in range(_NB)]
    dscr[pl.ds(0, _NB), :] = jnp.concatenate(dstep, axis=0)

    iota_s = lax.broadcasted_iota(jnp.int32, (_KNN, 1), 0)

    pen = jnp.zeros((_KNN, _ROWS), f32)
    for t in range(5):
        bit = ((iota_s >> t) & 1) == 1
        pen = pen + jnp.where(bit, dstep[t], jnp.zeros((1, _ROWS), f32))
    msk = lax.broadcasted_iota(jnp.int32, (_KNN, _ROWS), 0).astype(f32)
    for k in (2, 4, 8, 16, 32):
        asc = (iota_s & k) == 0
        s = k // 2
        while s >= 1:
            pen, msk = _cmpex(pen, msk, s, iota_s, asc)
            s //= 2

    rev_p = (lax.broadcasted_iota(jnp.int32, (_KNN, _KNN), 0)
             + lax.broadcasted_iota(jnp.int32, (_KNN, _KNN), 1)
             == _KNN - 1).astype(f32)

    def _rev32(z):
        return lax.dot_general(rev_p, z, (((1,), (0,)), ((), ())),
                               preferred_element_type=f32,
                         precision=lax.Precision.HIGHEST)

    asc_all = jnp.full((_KNN, 1), True)

    def _merge_step(t, carry):
        pen, msk, bitf = carry
        rp = _rev32(pen) + dscr[pl.ds(t, 1), :]
        rm = _rev32(msk) + bitf
        keep = pen <= rp
        pen = jnp.where(keep, pen, rp)
        msk = jnp.where(keep, msk, rm)
        for s in (16, 8, 4, 2, 1):
            pen, msk = _cmpex(pen, msk, s, iota_s, asc_all)
        return pen, msk, bitf * f32(2.0)

    pen, msk, _ = lax.fori_loop(
        5, _NB, _merge_step, (pen, msk, f32(float(1 << 5))))

    sc = best - pen
    sc = sc - jnp.max(sc, axis=0, keepdims=True)
    e = jnp.exp(sc)
    w = e / jnp.sum(e, axis=0, keepdims=True)
    idx = codes.astype(jnp.int32) ^ msk.astype(jnp.int32)

    idx_cat = jnp.concatenate(
        [idx[:, h * _SEQ:(h + 1) * _SEQ] for h in range(_H)], axis=0)
    w_cat = jnp.concatenate(
        [w[:, h * _SEQ:(h + 1) * _SEQ] for h in range(_H)], axis=0)
    idx_ref[...] = jnp.swapaxes(idx_cat, 0, 1)
    wT = jnp.swapaxes(w_cat, 0, 1)
    rep = (lax.broadcasted_iota(jnp.int32, (_KPT, _LANES * _KPT), 1) // _LANES
           == lax.broadcasted_iota(jnp.int32, (_KPT, _LANES * _KPT), 0)
           ).astype(f32)
    wbc_ref[...] = lax.dot_general(wT, rep, (((1,), (0,)), ((), ())),
                                   preferred_element_type=f32,
                         precision=lax.Precision.HIGHEST)


def _run_tc_a(xT, ktm, wqd, bqd2, wqu):
    return pl.pallas_call(
        _tc_query_beam,
        out_shape=[
            jax.ShapeDtypeStruct((_SEQ, _KPT), jnp.int32),
            jax.ShapeDtypeStruct((_SEQ, _LANES * _KPT), jnp.float32),
        ],
        scratch_shapes=[pltpu.VMEM((24, _ROWS), jnp.float32)],
    )(xT, ktm, wqd, bqd2, wqu)


_LANES = 16
_NW = 32
_TPW = _SEQ // _NW
_KPT = _H * _KNN


_WGRP = 8


_HKPT = _KPT // 2


def _sc_bag_body(idx_hbm, wbc_hbm, val_hbm, out_hbm,
                 idxblk, wbuf, rows, outacc,
                 sem_r0, sem_r1, sem_r2, sem_r3, sem_w0, sem_w1):
    f32 = jnp.float32
    wid = lax.axis_index("s") * 2 + lax.axis_index("c")
    base = wid * _TPW
    pltpu.sync_copy(idx_hbm.at[pl.ds(base, _TPW), :], idxblk)

    sem_r = (sem_r0, sem_r1, sem_r2, sem_r3)
    sem_w = (sem_w0, sem_w1)

    def start_half(b, t, j):
        pltpu.make_async_copy(
            val_hbm.at[idxblk.at[t, pl.ds(j * _HKPT, _HKPT)]],
            rows.at[b], sem_r[b]).start()

    def wait_half(b):
        pltpu.make_async_copy(
            val_hbm.at[idxblk.at[0, pl.ds(0, _HKPT)]],
            rows.at[b], sem_r[b]).wait()

    def start_wgrp(sub, g):
        pltpu.make_async_copy(
            wbc_hbm.at[pl.ds(base + g * _WGRP, _WGRP), :],
            wbuf.at[sub], sem_w[sub]).start()

    def wait_wgrp(sub):
        pltpu.make_async_copy(
            wbc_hbm.at[pl.ds(base, _WGRP), :],
            wbuf.at[sub], sem_w[sub]).wait()

    def accum_half(sub, i, b, half, accs):
        def kbody(k, accs):
            wb = wbuf[sub, i, pl.ds((half * _HKPT + k) * _LANES, _LANES)]
            return tuple(
                accs[j] + wb * rows[b, k, pl.ds(j * _LANES, _LANES)]
                for j in range(_VD // _LANES))
        return lax.fori_loop(0, _HKPT, kbody, accs)

    start_wgrp(0, 0)
    start_wgrp(1, 1)
    for p in range(2):
        for j in range(2):
            start_half(2 * p + j, p, j)

    def gpbody(gp, carry):
        for sub in range(2):
            g = gp * 2 + sub
            wait_wgrp(sub)
            for i in range(_WGRP):
                t = g * _WGRP + i
                p = i % 2
                accs = tuple(jnp.zeros((_LANES,), f32)
                             for _ in range(_VD // _LANES))
                for j in range(2):
                    wait_half(2 * p + j)
                    accs = accum_half(sub, i, 2 * p + j, j, accs)
                for j in range(_VD // _LANES):
                    outacc[t, pl.ds(j * _LANES, _LANES)] = accs[j]

                @pl.when(t + 2 < _TPW)
                def _pf():
                    for j in range(2):
                        start_half(2 * p + j, t + 2, j)

            @pl.when(g + 2 < _TPW // _WGRP)
            def _pfw():
                start_wgrp(sub, g + 2)
        return carry

    lax.fori_loop(0, _TPW // (2 * _WGRP), gpbody, 0)
    pltpu.sync_copy(outacc, out_hbm.at[pl.ds(base, _TPW), :])


def _run_sc_bag(idx, wbc, values):
    mesh = plsc.VectorSubcoreMesh(core_axis_name="c", subcore_axis_name="s")
    kern = functools.partial(
        pl.kernel,
        mesh=mesh,
        out_type=jax.ShapeDtypeStruct((_SEQ, _VD), jnp.float32),
        scratch_types=[
            pltpu.VMEM((_TPW, _KPT), jnp.int32),
            pltpu.VMEM((2, _WGRP, _LANES * _KPT), jnp.float32),
            pltpu.VMEM((4, _HKPT, _VD), jnp.float32),
            pltpu.VMEM((_TPW, _VD), jnp.float32),
            pltpu.SemaphoreType.DMA,
            pltpu.SemaphoreType.DMA,
            pltpu.SemaphoreType.DMA,
            pltpu.SemaphoreType.DMA,
            pltpu.SemaphoreType.DMA,
            pltpu.SemaphoreType.DMA,
        ],
    )(_sc_bag_body)
    return kern(idx, wbc, values)


def _tc_proj_body(y0_ref, wvp_ref, o_ref):
    o_ref[...] = lax.dot_general(
        y0_ref[...], wvp_ref[...], (((1,), (1,)), ((), ())),
        preferred_element_type=jnp.float32)


def _run_tc_c(y0, wvp):
    return pl.pallas_call(
        _tc_proj_body,
        out_shape=jax.ShapeDtypeStruct((_SEQ, _D_MODEL), jnp.float32),
    )(y0, wvp)


_COLJ = np.arange(_KD * _H) // _BD
_ROWJ = np.arange(2 * _H * _NB) % (_H * _NB)
_KMASK = (_ROWJ[:, None] == _COLJ[None, :]).astype(np.float32)


def kernel(x, keys_p, values, Wqd, bqd, Wqu, Wvp):
    bsz, seq_len, _ = x.shape
    xT = jnp.swapaxes(x.reshape(bsz * seq_len, _D_MODEL), 0, 1)
    keys_r = jnp.transpose(keys_p, (2, 0, 1, 3)).reshape(2 * _H * _NB, _BD)
    ktm = jnp.tile(keys_r, (1, _H * _NB)) * _KMASK
    bqd2 = bqd.reshape(_QR, 1)
    idx, wbc = _run_tc_a(xT, ktm, Wqd, bqd2, Wqu)
    y0 = _run_sc_bag(idx, wbc, values)
    y = _run_tc_c(y0, Wvp)
    return y.reshape(bsz, seq_len, _D_MODEL)
